# Initial kernel scaffold; baseline (speedup 1.0000x reference)
import jax
import jax.numpy as jnp
from jax.experimental import pallas as pl


def kernel(edge_index, cj, ci, u_feat, i_feat, weight):
    raise NotImplementedError("write your pallas kernel here")


# speedup vs baseline: 8.2184x; 8.2184x over previous
import functools

import jax
import jax.numpy as jnp
from jax import lax
from jax.experimental import pallas as pl
from jax.experimental.pallas import tpu as pltpu
from jax.experimental.pallas import tpu_sc as plsc

N = 10000
F = 128
E = 320000

NC = 2
NS = 16
EDGES_PER_TILE = E // (NC * NS)
B = 125
NBLK = EDGES_PER_TILE // B
EBLK_TOTAL = E // B
ROWS_MAIN = 624
TAIL0 = NS * ROWS_MAIN
ZROWS = 16


def _sc_body(feat_hbm, src_hbm, dst_hbm, out_hbm,
             src_idx, dst_idx, rows, zbuf, acc, sem):
    c = lax.axis_index("c")
    s = lax.axis_index("s")
    wid = c * NS + s

    zv = jnp.zeros((16,), jnp.float32)
    for r in range(ZROWS):
        for q in range(F // 16):
            zbuf[r, pl.ds(q * 16, 16)] = zv
    row0 = s * ROWS_MAIN
    for t in range(ROWS_MAIN // ZROWS):
        pltpu.sync_copy(zbuf, acc.at[pl.ds(row0 + t * ZROWS, ZROWS)])

    @pl.when(s == NS - 1)
    def _():
        pltpu.sync_copy(zbuf, acc.at[pl.ds(TAIL0, N - TAIL0)])

    plsc.subcore_barrier()

    blk0 = wid * NBLK
    pltpu.sync_copy(src_hbm.at[pl.ds(blk0, NBLK)], src_idx)
    pltpu.sync_copy(dst_hbm.at[pl.ds(blk0, NBLK)], dst_idx)

    def body(i, carry):
        pltpu.async_copy(feat_hbm.at[src_idx.at[i]], rows, sem).wait()
        pltpu.sync_copy(rows, acc.at[dst_idx.at[i]], add=True)
        return carry

    lax.fori_loop(0, NBLK, body, 0)
    plsc.subcore_barrier()

    pltpu.sync_copy(acc.at[pl.ds(row0, ROWS_MAIN)],
                    out_hbm.at[pl.ds(c * N + row0, ROWS_MAIN)])

    @pl.when(s == NS - 1)
    def _():
        pltpu.sync_copy(acc.at[pl.ds(TAIL0, N - TAIL0)],
                        out_hbm.at[pl.ds(c * N + TAIL0, N - TAIL0)])


_sc_gather_scatter = functools.partial(
    pl.kernel,
    mesh=plsc.VectorSubcoreMesh(core_axis_name="c", subcore_axis_name="s"),
    out_type=jax.ShapeDtypeStruct((NC * N, F), jnp.float32),
    scratch_types=[
        pltpu.VMEM((NBLK, B), jnp.int32),
        pltpu.VMEM((NBLK, B), jnp.int32),
        pltpu.VMEM((B, F), jnp.float32),
        pltpu.VMEM((ZROWS, F), jnp.float32),
        pltpu.VMEM_SHARED((N, F), jnp.float32),
        pltpu.SemaphoreType.DMA,
    ],
)(_sc_body)


def _feat_body(w_ref, cj_ref, o_ref):
    o_ref[...] = w_ref[...] * cj_ref[...]


def _comb_body(p_ref, ci_ref, o_ref):
    o_ref[...] = (p_ref[0] + p_ref[1]) * ci_ref[...]


def kernel(edge_index, cj, ci, u_feat, i_feat, weight):
    src = edge_index[0].astype(jnp.int32).reshape(EBLK_TOTAL, B)
    dst = edge_index[1].astype(jnp.int32).reshape(EBLK_TOTAL, B)

    feat = pl.pallas_call(
        _feat_body,
        out_shape=jax.ShapeDtypeStruct((N, F), jnp.float32),
        grid=(10,),
        in_specs=[pl.BlockSpec((N // 10, F), lambda i: (i, 0)),
                  pl.BlockSpec((N // 10, 1), lambda i: (i, 0))],
        out_specs=pl.BlockSpec((N // 10, F), lambda i: (i, 0)),
    )(weight, cj)

    partials = _sc_gather_scatter(feat, src, dst)
    partials = partials.reshape(NC, N, F)

    rst = pl.pallas_call(
        _comb_body,
        out_shape=jax.ShapeDtypeStruct((N, F), jnp.float32),
        grid=(10,),
        in_specs=[pl.BlockSpec((NC, N // 10, F), lambda i: (0, i, 0)),
                  pl.BlockSpec((N // 10, 1), lambda i: (i, 0))],
        out_specs=pl.BlockSpec((N // 10, F), lambda i: (i, 0)),
    )(partials, ci)
    return rst


# speedup vs baseline: 10.9048x; 1.3269x over previous
import functools

import jax
import jax.numpy as jnp
from jax import lax
from jax.experimental import pallas as pl
from jax.experimental.pallas import tpu as pltpu
from jax.experimental.pallas import tpu_sc as plsc

N = 10000
F = 128
E = 320000

NC = 2
NS = 16
EDGES_PER_TILE = E // (NC * NS)
B = 125
NBLK = EDGES_PER_TILE // B
CHUNK = 16
EBLK_TOTAL = E // B
ROWS_MAIN = 624
TAIL0 = NS * ROWS_MAIN
ZROWS = 16


def _sc_body(feat_hbm, src_hbm, dst_hbm, out_hbm,
             src_idx, dst_idx, rows, zbuf, acc, sem):
    c = lax.axis_index("c")
    s = lax.axis_index("s")
    wid = c * NS + s

    zv = jnp.zeros((16,), jnp.float32)
    for r in range(ZROWS):
        for q in range(F // 16):
            zbuf[r, pl.ds(q * 16, 16)] = zv
    row0 = s * ROWS_MAIN
    for t in range(ROWS_MAIN // ZROWS):
        pltpu.sync_copy(zbuf, acc.at[pl.ds(row0 + t * ZROWS, ZROWS)])

    @pl.when(s == NS - 1)
    def _():
        pltpu.sync_copy(zbuf, acc.at[pl.ds(TAIL0, N - TAIL0)])

    plsc.subcore_barrier()

    blk0 = wid * NBLK
    rows0 = rows.at[0]
    rows1 = rows.at[1]
    sem0 = sem.at[0]
    sem1 = sem.at[1]

    def chunk_body(ch, carry):
        pltpu.sync_copy(src_hbm.at[pl.ds(blk0 + ch * CHUNK, CHUNK)], src_idx)
        pltpu.sync_copy(dst_hbm.at[pl.ds(blk0 + ch * CHUNK, CHUNK)], dst_idx)
        pltpu.async_copy(feat_hbm.at[src_idx.at[0]], rows0, sem0)

        def body(j, c2):
            i0 = 2 * j
            pltpu.async_copy(feat_hbm.at[src_idx.at[i0 + 1]], rows1, sem1)
            pltpu.make_async_copy(feat_hbm.at[src_idx.at[i0]], rows0, sem0).wait()
            pltpu.sync_copy(rows0, acc.at[dst_idx.at[i0]], add=True)

            @pl.when(j + 1 < CHUNK // 2)
            def _():
                pltpu.async_copy(feat_hbm.at[src_idx.at[i0 + 2]], rows0, sem0)

            pltpu.make_async_copy(feat_hbm.at[src_idx.at[i0 + 1]], rows1, sem1).wait()
            pltpu.sync_copy(rows1, acc.at[dst_idx.at[i0 + 1]], add=True)
            return c2

        lax.fori_loop(0, CHUNK // 2, body, 0)
        return carry

    lax.fori_loop(0, NBLK // CHUNK, chunk_body, 0)
    plsc.subcore_barrier()

    pltpu.sync_copy(acc.at[pl.ds(row0, ROWS_MAIN)],
                    out_hbm.at[pl.ds(c * N + row0, ROWS_MAIN)])

    @pl.when(s == NS - 1)
    def _():
        pltpu.sync_copy(acc.at[pl.ds(TAIL0, N - TAIL0)],
                        out_hbm.at[pl.ds(c * N + TAIL0, N - TAIL0)])


_sc_gather_scatter = functools.partial(
    pl.kernel,
    mesh=plsc.VectorSubcoreMesh(core_axis_name="c", subcore_axis_name="s"),
    out_type=jax.ShapeDtypeStruct((NC * N, F), jnp.float32),
    scratch_types=[
        pltpu.VMEM((CHUNK, B), jnp.int32),
        pltpu.VMEM((CHUNK, B), jnp.int32),
        pltpu.VMEM((2, B, F), jnp.float32),
        pltpu.VMEM((ZROWS, F), jnp.float32),
        pltpu.VMEM_SHARED((N, F), jnp.float32),
        pltpu.SemaphoreType.DMA((2,)),
    ],
)(_sc_body)


def _feat_body(w_ref, cj_ref, o_ref):
    o_ref[...] = w_ref[...] * cj_ref[...]


def _comb_body(p_ref, ci_ref, o_ref):
    o_ref[...] = (p_ref[0] + p_ref[1]) * ci_ref[...]


def kernel(edge_index, cj, ci, u_feat, i_feat, weight):
    src = edge_index[0].astype(jnp.int32).reshape(EBLK_TOTAL, B)
    dst = edge_index[1].astype(jnp.int32).reshape(EBLK_TOTAL, B)

    feat = pl.pallas_call(
        _feat_body,
        out_shape=jax.ShapeDtypeStruct((N, F), jnp.float32),
        grid=(10,),
        in_specs=[pl.BlockSpec((N // 10, F), lambda i: (i, 0)),
                  pl.BlockSpec((N // 10, 1), lambda i: (i, 0))],
        out_specs=pl.BlockSpec((N // 10, F), lambda i: (i, 0)),
    )(weight, cj)

    partials = _sc_gather_scatter(feat, src, dst)
    partials = partials.reshape(NC, N, F)

    rst = pl.pallas_call(
        _comb_body,
        out_shape=jax.ShapeDtypeStruct((N, F), jnp.float32),
        grid=(10,),
        in_specs=[pl.BlockSpec((NC, N // 10, F), lambda i: (0, i, 0)),
                  pl.BlockSpec((N // 10, 1), lambda i: (i, 0))],
        out_specs=pl.BlockSpec((N // 10, F), lambda i: (i, 0)),
    )(partials, ci)
    return rst


# speedup vs baseline: 11.6488x; 1.0682x over previous
import functools

import jax
import jax.numpy as jnp
from jax import lax
from jax.experimental import pallas as pl
from jax.experimental.pallas import tpu as pltpu
from jax.experimental.pallas import tpu_sc as plsc

N = 10000
F = 128
E = 320000

NC = 2
NS = 16
EDGES_PER_TILE = E // (NC * NS)
B = 125
NBLK = EDGES_PER_TILE // B
CHUNK = 16
EBLK_TOTAL = E // B
ROWS_MAIN = 624
TAIL0 = NS * ROWS_MAIN
ZROWS = 16


def _sc_body(feat_hbm, edges_hbm, out_hbm,
             src_idx, dst_idx, rows, zbuf, acc, sem):
    c = lax.axis_index("c")
    s = lax.axis_index("s")
    wid = c * NS + s

    zv = jnp.zeros((16,), jnp.float32)
    for r in range(ZROWS):
        for q in range(F // 16):
            zbuf[r, pl.ds(q * 16, 16)] = zv
    row0 = s * ROWS_MAIN
    for t in range(ROWS_MAIN // ZROWS):
        pltpu.sync_copy(zbuf, acc.at[pl.ds(row0 + t * ZROWS, ZROWS)])

    @pl.when(s == NS - 1)
    def _():
        pltpu.sync_copy(zbuf, acc.at[pl.ds(TAIL0, N - TAIL0)])

    plsc.subcore_barrier()

    blk0 = wid * NBLK
    rows0 = rows.at[0]
    rows1 = rows.at[1]
    sem0 = sem.at[0]
    sem1 = sem.at[1]

    def chunk_body(ch, carry):
        pltpu.sync_copy(edges_hbm.at[0, pl.ds(blk0 + ch * CHUNK, CHUNK)], src_idx)
        pltpu.sync_copy(edges_hbm.at[1, pl.ds(blk0 + ch * CHUNK, CHUNK)], dst_idx)
        pltpu.async_copy(feat_hbm.at[src_idx.at[0]], rows0, sem0)

        def body(j, c2):
            i0 = 2 * j
            pltpu.async_copy(feat_hbm.at[src_idx.at[i0 + 1]], rows1, sem1)
            pltpu.make_async_copy(feat_hbm.at[src_idx.at[i0]], rows0, sem0).wait()
            pltpu.sync_copy(rows0, acc.at[dst_idx.at[i0]], add=True)

            @pl.when(j + 1 < CHUNK // 2)
            def _():
                pltpu.async_copy(feat_hbm.at[src_idx.at[i0 + 2]], rows0, sem0)

            pltpu.make_async_copy(feat_hbm.at[src_idx.at[i0 + 1]], rows1, sem1).wait()
            pltpu.sync_copy(rows1, acc.at[dst_idx.at[i0 + 1]], add=True)
            return c2

        lax.fori_loop(0, CHUNK // 2, body, 0)
        return carry

    lax.fori_loop(0, NBLK // CHUNK, chunk_body, 0)
    plsc.subcore_barrier()

    pltpu.sync_copy(acc.at[pl.ds(row0, ROWS_MAIN)],
                    out_hbm.at[pl.ds(c * N + row0, ROWS_MAIN)])

    @pl.when(s == NS - 1)
    def _():
        pltpu.sync_copy(acc.at[pl.ds(TAIL0, N - TAIL0)],
                        out_hbm.at[pl.ds(c * N + TAIL0, N - TAIL0)])


_sc_gather_scatter = functools.partial(
    pl.kernel,
    mesh=plsc.VectorSubcoreMesh(core_axis_name="c", subcore_axis_name="s"),
    out_type=jax.ShapeDtypeStruct((NC * N, F), jnp.float32),
    scratch_types=[
        pltpu.VMEM((CHUNK, B), jnp.int32),
        pltpu.VMEM((CHUNK, B), jnp.int32),
        pltpu.VMEM((2, B, F), jnp.float32),
        pltpu.VMEM((ZROWS, F), jnp.float32),
        pltpu.VMEM_SHARED((N, F), jnp.float32),
        pltpu.SemaphoreType.DMA((2,)),
    ],
)(_sc_body)


def _feat_body(w_ref, cj_ref, o_ref):
    o_ref[...] = w_ref[...] * cj_ref[...]


def _comb_body(p_ref, ci_ref, o_ref):
    o_ref[...] = (p_ref[0] + p_ref[1]) * ci_ref[...]


def kernel(edge_index, cj, ci, u_feat, i_feat, weight):
    edges = edge_index.astype(jnp.int32).reshape(2, EBLK_TOTAL, B)

    feat = pl.pallas_call(
        _feat_body,
        out_shape=jax.ShapeDtypeStruct((N, F), jnp.float32),
        grid=(10,),
        in_specs=[pl.BlockSpec((N // 10, F), lambda i: (i, 0)),
                  pl.BlockSpec((N // 10, 1), lambda i: (i, 0))],
        out_specs=pl.BlockSpec((N // 10, F), lambda i: (i, 0)),
    )(weight, cj)

    partials = _sc_gather_scatter(feat, edges)
    partials = partials.reshape(NC, N, F)

    rst = pl.pallas_call(
        _comb_body,
        out_shape=jax.ShapeDtypeStruct((N, F), jnp.float32),
        grid=(10,),
        in_specs=[pl.BlockSpec((NC, N // 10, F), lambda i: (0, i, 0)),
                  pl.BlockSpec((N // 10, 1), lambda i: (i, 0))],
        out_specs=pl.BlockSpec((N // 10, F), lambda i: (i, 0)),
    )(partials, ci)
    return rst


# speedup vs baseline: 11.9994x; 1.0301x over previous
import functools

import jax
import jax.numpy as jnp
from jax import lax
from jax.experimental import pallas as pl
from jax.experimental.pallas import tpu as pltpu
from jax.experimental.pallas import tpu_sc as plsc

N = 10000
F = 128
E = 320000

NC = 2
NS = 16
EDGES_PER_TILE = E // (NC * NS)
B = 125
NBLK = EDGES_PER_TILE // B
CHUNK = 16
EBLK_TOTAL = E // B
ROWS_MAIN = 624
TAIL0 = NS * ROWS_MAIN
ZROWS = 16


def _sc_body(feat_hbm, edges_hbm, out_hbm,
             src_idx, dst_idx, rows, zbuf, acc, sem):
    c = lax.axis_index("c")
    s = lax.axis_index("s")
    wid = c * NS + s

    zv = jnp.zeros((16,), jnp.float32)
    for r in range(ZROWS):
        for q in range(F // 16):
            zbuf[r, pl.ds(q * 16, 16)] = zv
    row0 = s * ROWS_MAIN
    for t in range(ROWS_MAIN // ZROWS):
        pltpu.sync_copy(zbuf, acc.at[pl.ds(row0 + t * ZROWS, ZROWS)])

    @pl.when(s == NS - 1)
    def _():
        pltpu.sync_copy(zbuf, acc.at[pl.ds(TAIL0, N - TAIL0)])

    plsc.subcore_barrier()

    blk0 = wid * NBLK
    rows0 = rows.at[0]
    rows1 = rows.at[1]
    sem0 = sem.at[0]
    sem1 = sem.at[1]

    def chunk_body(ch, carry):
        pltpu.sync_copy(edges_hbm.at[0, pl.ds(blk0 + ch * CHUNK, CHUNK)], src_idx)
        pltpu.sync_copy(edges_hbm.at[1, pl.ds(blk0 + ch * CHUNK, CHUNK)], dst_idx)
        pltpu.async_copy(feat_hbm.at[src_idx.at[0]], rows0, sem0)

        def body(j, c2):
            i0 = 2 * j
            pltpu.async_copy(feat_hbm.at[src_idx.at[i0 + 1]], rows1, sem1)
            pltpu.make_async_copy(feat_hbm.at[src_idx.at[i0]], rows0, sem0).wait()
            pltpu.sync_copy(rows0, acc.at[dst_idx.at[i0]], add=True)

            @pl.when(j + 1 < CHUNK // 2)
            def _():
                pltpu.async_copy(feat_hbm.at[src_idx.at[i0 + 2]], rows0, sem0)

            pltpu.make_async_copy(feat_hbm.at[src_idx.at[i0 + 1]], rows1, sem1).wait()
            pltpu.sync_copy(rows1, acc.at[dst_idx.at[i0 + 1]], add=True)
            return c2

        lax.fori_loop(0, CHUNK // 2, body, 0)
        return carry

    lax.fori_loop(0, NBLK // CHUNK, chunk_body, 0)
    plsc.subcore_barrier()

    pltpu.sync_copy(acc.at[pl.ds(row0, ROWS_MAIN)],
                    out_hbm.at[pl.ds(c * N + row0, ROWS_MAIN)])

    @pl.when(s == NS - 1)
    def _():
        pltpu.sync_copy(acc.at[pl.ds(TAIL0, N - TAIL0)],
                        out_hbm.at[pl.ds(c * N + TAIL0, N - TAIL0)])


_sc_gather_scatter = functools.partial(
    pl.kernel,
    mesh=plsc.VectorSubcoreMesh(core_axis_name="c", subcore_axis_name="s"),
    out_type=jax.ShapeDtypeStruct((NC * N, F), jnp.float32),
    scratch_types=[
        pltpu.VMEM((CHUNK, B), jnp.int32),
        pltpu.VMEM((CHUNK, B), jnp.int32),
        pltpu.VMEM((2, B, F), jnp.float32),
        pltpu.VMEM((ZROWS, F), jnp.float32),
        pltpu.VMEM_SHARED((N, F), jnp.float32),
        pltpu.SemaphoreType.DMA((2,)),
    ],
)(_sc_body)


def _feat_body(w_ref, cj_ref, o_ref):
    o_ref[...] = w_ref[...] * cj_ref[...]


def _comb_body(p_ref, ci_ref, o_ref):
    o_ref[...] = (p_ref[0] + p_ref[1]) * ci_ref[...]


def kernel(edge_index, cj, ci, u_feat, i_feat, weight):
    edges = edge_index.astype(jnp.int32).reshape(2, EBLK_TOTAL, B)

    feat = pl.pallas_call(
        _feat_body,
        out_shape=jax.ShapeDtypeStruct((N, F), jnp.float32),
    )(weight, cj)

    partials = _sc_gather_scatter(feat, edges)
    partials = partials.reshape(NC, N, F)

    rst = pl.pallas_call(
        _comb_body,
        out_shape=jax.ShapeDtypeStruct((N, F), jnp.float32),
    )(partials, ci)
    return rst


# speedup vs baseline: 12.4045x; 1.0338x over previous
import functools

import jax
import jax.numpy as jnp
from jax import lax
from jax.experimental import pallas as pl
from jax.experimental.pallas import tpu as pltpu
from jax.experimental.pallas import tpu_sc as plsc

N = 10000
F = 128
E = 320000

NC = 2
NS = 16
EDGES_PER_TILE = E // (NC * NS)
B = 125
NBLK = EDGES_PER_TILE // B
CHUNK = 16
EBLK_TOTAL = E // B
ROWS_MAIN = 624
TAIL0 = NS * ROWS_MAIN
ZROWS = 16


def _sc_body(feat_hbm, edges_hbm, out_hbm,
             src_idx, dst_idx, rows, zbuf, acc, sem, isem):
    c = lax.axis_index("c")
    s = lax.axis_index("s")
    wid = c * NS + s

    zv = jnp.zeros((16,), jnp.float32)
    for r in range(ZROWS):
        for q in range(F // 16):
            zbuf[r, pl.ds(q * 16, 16)] = zv
    row0 = s * ROWS_MAIN
    for t in range(ROWS_MAIN // ZROWS):
        pltpu.sync_copy(zbuf, acc.at[pl.ds(row0 + t * ZROWS, ZROWS)])

    @pl.when(s == NS - 1)
    def _():
        pltpu.sync_copy(zbuf, acc.at[pl.ds(TAIL0, N - TAIL0)])

    plsc.subcore_barrier()

    blk0 = wid * NBLK
    rows0 = rows.at[0]
    rows1 = rows.at[1]
    sem0 = sem.at[0]
    sem1 = sem.at[1]
    NCH = NBLK // CHUNK

    def idx_start(ch, p):
        off = blk0 + ch * CHUNK
        pltpu.async_copy(edges_hbm.at[0, pl.ds(off, CHUNK)], src_idx.at[p],
                         isem.at[0])
        pltpu.async_copy(edges_hbm.at[1, pl.ds(off, CHUNK)], dst_idx.at[p],
                         isem.at[1])

    def idx_wait(ch, p):
        off = blk0 + ch * CHUNK
        pltpu.make_async_copy(edges_hbm.at[0, pl.ds(off, CHUNK)],
                              src_idx.at[p], isem.at[0]).wait()
        pltpu.make_async_copy(edges_hbm.at[1, pl.ds(off, CHUNK)],
                              dst_idx.at[p], isem.at[1]).wait()

    idx_start(0, 0)
    for ch in range(NCH):
        p = ch % 2
        idx_wait(ch, p)
        if ch + 1 < NCH:
            idx_start(ch + 1, 1 - p)
        sidx = src_idx.at[p]
        didx = dst_idx.at[p]
        pltpu.async_copy(feat_hbm.at[sidx.at[0]], rows0, sem0)

        def body(j, c2, sidx=sidx, didx=didx):
            i0 = 2 * j
            pltpu.async_copy(feat_hbm.at[sidx.at[i0 + 1]], rows1, sem1)
            pltpu.make_async_copy(feat_hbm.at[sidx.at[i0]], rows0, sem0).wait()
            pltpu.sync_copy(rows0, acc.at[didx.at[i0]], add=True)

            @pl.when(j + 1 < CHUNK // 2)
            def _():
                pltpu.async_copy(feat_hbm.at[sidx.at[i0 + 2]], rows0, sem0)

            pltpu.make_async_copy(feat_hbm.at[sidx.at[i0 + 1]], rows1,
                                  sem1).wait()
            pltpu.sync_copy(rows1, acc.at[didx.at[i0 + 1]], add=True)
            return c2

        lax.fori_loop(0, CHUNK // 2, body, 0)
    plsc.subcore_barrier()

    pltpu.sync_copy(acc.at[pl.ds(row0, ROWS_MAIN)],
                    out_hbm.at[pl.ds(c * N + row0, ROWS_MAIN)])

    @pl.when(s == NS - 1)
    def _():
        pltpu.sync_copy(acc.at[pl.ds(TAIL0, N - TAIL0)],
                        out_hbm.at[pl.ds(c * N + TAIL0, N - TAIL0)])


_sc_gather_scatter = functools.partial(
    pl.kernel,
    mesh=plsc.VectorSubcoreMesh(core_axis_name="c", subcore_axis_name="s"),
    out_type=jax.ShapeDtypeStruct((NC * N, F), jnp.float32),
    scratch_types=[
        pltpu.VMEM((2, CHUNK, B), jnp.int32),
        pltpu.VMEM((2, CHUNK, B), jnp.int32),
        pltpu.VMEM((2, B, F), jnp.float32),
        pltpu.VMEM((ZROWS, F), jnp.float32),
        pltpu.VMEM_SHARED((N, F), jnp.float32),
        pltpu.SemaphoreType.DMA((2,)),
        pltpu.SemaphoreType.DMA((2,)),
    ],
)(_sc_body)


def _feat_body(w_ref, cj_ref, o_ref):
    o_ref[...] = w_ref[...] * cj_ref[...]


def _comb_body(p_ref, ci_ref, o_ref):
    o_ref[...] = (p_ref[0] + p_ref[1]) * ci_ref[...]


def kernel(edge_index, cj, ci, u_feat, i_feat, weight):
    edges = edge_index.astype(jnp.int32).reshape(2, EBLK_TOTAL, B)

    feat = pl.pallas_call(
        _feat_body,
        out_shape=jax.ShapeDtypeStruct((N, F), jnp.float32),
    )(weight, cj)

    partials = _sc_gather_scatter(feat, edges)
    partials = partials.reshape(NC, N, F)

    rst = pl.pallas_call(
        _comb_body,
        out_shape=jax.ShapeDtypeStruct((N, F), jnp.float32),
    )(partials, ci)
    return rst


# speedup vs baseline: 12.6469x; 1.0195x over previous
import functools

import jax
import jax.numpy as jnp
from jax import lax
from jax.experimental import pallas as pl
from jax.experimental.pallas import tpu as pltpu
from jax.experimental.pallas import tpu_sc as plsc

N = 10000
F = 128
E = 320000

NC = 2
NS = 16
EDGES_PER_TILE = E // (NC * NS)
B = 125
NBLK = EDGES_PER_TILE // B
CHUNK = 16
EBLK_TOTAL = E // B
ROWS_MAIN = 624
TAIL0 = NS * ROWS_MAIN
ZROWS = 48


def _sc_body(feat_hbm, edges_hbm, out_hbm,
             src_idx, dst_idx, rows, zbuf, acc, sem, isem):
    c = lax.axis_index("c")
    s = lax.axis_index("s")
    wid = c * NS + s
    blk0 = wid * NBLK

    def idx_start(ch, p):
        off = blk0 + ch * CHUNK
        pltpu.async_copy(edges_hbm.at[0, pl.ds(off, CHUNK)], src_idx.at[p],
                         isem.at[0])
        pltpu.async_copy(edges_hbm.at[1, pl.ds(off, CHUNK)], dst_idx.at[p],
                         isem.at[1])

    def idx_wait(ch, p):
        off = blk0 + ch * CHUNK
        pltpu.make_async_copy(edges_hbm.at[0, pl.ds(off, CHUNK)],
                              src_idx.at[p], isem.at[0]).wait()
        pltpu.make_async_copy(edges_hbm.at[1, pl.ds(off, CHUNK)],
                              dst_idx.at[p], isem.at[1]).wait()

    idx_start(0, 0)

    zv = jnp.zeros((16,), jnp.float32)
    for r in range(ZROWS):
        for q in range(F // 16):
            zbuf[r, pl.ds(q * 16, 16)] = zv
    row0 = s * ROWS_MAIN
    for t in range(ROWS_MAIN // ZROWS):
        pltpu.async_copy(zbuf, acc.at[pl.ds(row0 + t * ZROWS, ZROWS)],
                         sem.at[0])
    for t in range(ROWS_MAIN // ZROWS):
        pltpu.make_async_copy(zbuf, acc.at[pl.ds(row0 + t * ZROWS, ZROWS)],
                              sem.at[0]).wait()

    @pl.when(s == NS - 1)
    def _():
        pltpu.sync_copy(zbuf.at[pl.ds(0, N - TAIL0)],
                        acc.at[pl.ds(TAIL0, N - TAIL0)])

    idx_wait(0, 0)
    pltpu.async_copy(feat_hbm.at[src_idx.at[0].at[0]], rows.at[0], sem.at[0])
    plsc.subcore_barrier()

    blk0 = wid * NBLK
    rows0 = rows.at[0]
    rows1 = rows.at[1]
    sem0 = sem.at[0]
    sem1 = sem.at[1]
    NCH = NBLK // CHUNK

    for ch in range(NCH):
        p = ch % 2
        if ch > 0:
            idx_wait(ch, p)
        if ch + 1 < NCH:
            idx_start(ch + 1, 1 - p)
        sidx = src_idx.at[p]
        didx = dst_idx.at[p]
        if ch > 0:
            pltpu.async_copy(feat_hbm.at[sidx.at[0]], rows0, sem0)

        def body(j, c2, sidx=sidx, didx=didx):
            i0 = 2 * j
            pltpu.async_copy(feat_hbm.at[sidx.at[i0 + 1]], rows1, sem1)
            pltpu.make_async_copy(feat_hbm.at[sidx.at[i0]], rows0, sem0).wait()
            pltpu.sync_copy(rows0, acc.at[didx.at[i0]], add=True)

            @pl.when(j + 1 < CHUNK // 2)
            def _():
                pltpu.async_copy(feat_hbm.at[sidx.at[i0 + 2]], rows0, sem0)

            pltpu.make_async_copy(feat_hbm.at[sidx.at[i0 + 1]], rows1,
                                  sem1).wait()
            pltpu.sync_copy(rows1, acc.at[didx.at[i0 + 1]], add=True)
            return c2

        lax.fori_loop(0, CHUNK // 2, body, 0)
    plsc.subcore_barrier()

    pltpu.sync_copy(acc.at[pl.ds(row0, ROWS_MAIN)],
                    out_hbm.at[pl.ds(c * N + row0, ROWS_MAIN)])

    @pl.when(s == NS - 1)
    def _():
        pltpu.sync_copy(acc.at[pl.ds(TAIL0, N - TAIL0)],
                        out_hbm.at[pl.ds(c * N + TAIL0, N - TAIL0)])


_sc_gather_scatter = functools.partial(
    pl.kernel,
    mesh=plsc.VectorSubcoreMesh(core_axis_name="c", subcore_axis_name="s"),
    out_type=jax.ShapeDtypeStruct((NC * N, F), jnp.float32),
    scratch_types=[
        pltpu.VMEM((2, CHUNK, B), jnp.int32),
        pltpu.VMEM((2, CHUNK, B), jnp.int32),
        pltpu.VMEM((2, B, F), jnp.float32),
        pltpu.VMEM((ZROWS, F), jnp.float32),
        pltpu.VMEM_SHARED((N, F), jnp.float32),
        pltpu.SemaphoreType.DMA((2,)),
        pltpu.SemaphoreType.DMA((2,)),
    ],
)(_sc_body)


def _feat_body(w_ref, cj_ref, o_ref):
    o_ref[...] = w_ref[...] * cj_ref[...]


def _comb_body(p_ref, ci_ref, o_ref):
    o_ref[...] = (p_ref[0] + p_ref[1]) * ci_ref[...]


def kernel(edge_index, cj, ci, u_feat, i_feat, weight):
    edges = edge_index.astype(jnp.int32).reshape(2, EBLK_TOTAL, B)

    feat = pl.pallas_call(
        _feat_body,
        out_shape=jax.ShapeDtypeStruct((N, F), jnp.float32),
    )(weight, cj)

    partials = _sc_gather_scatter(feat, edges)
    partials = partials.reshape(NC, N, F)

    rst = pl.pallas_call(
        _comb_body,
        out_shape=jax.ShapeDtypeStruct((N, F), jnp.float32),
    )(partials, ci)
    return rst


# speedup vs baseline: 13.1459x; 1.0395x over previous
import functools

import jax
import jax.numpy as jnp
from jax import lax
from jax.experimental import pallas as pl
from jax.experimental.pallas import tpu as pltpu
from jax.experimental.pallas import tpu_sc as plsc

N = 10000
F = 128
E = 320000

NC = 2
NS = 16
EDGES_PER_TILE = E // (NC * NS)
B = 125
NBLK = EDGES_PER_TILE // B
CHUNK = 16
EBLK_TOTAL = E // B
ROWS_MAIN = 624
TAIL0 = NS * ROWS_MAIN
ZROWS = 48


def _sc_body(feat_hbm, edges_hbm, out_hbm,
             src_idx, dst_idx, rows, zbuf, acc, sem, isem):
    c = lax.axis_index("c")
    s = lax.axis_index("s")
    wid = c * NS + s
    blk0 = wid * NBLK

    def idx_start(ch, p):
        off = blk0 + ch * CHUNK
        pltpu.async_copy(edges_hbm.at[0, pl.ds(off, CHUNK)], src_idx.at[p],
                         isem.at[0])
        pltpu.async_copy(edges_hbm.at[1, pl.ds(off, CHUNK)], dst_idx.at[p],
                         isem.at[1])

    def idx_wait(ch, p):
        off = blk0 + ch * CHUNK
        pltpu.make_async_copy(edges_hbm.at[0, pl.ds(off, CHUNK)],
                              src_idx.at[p], isem.at[0]).wait()
        pltpu.make_async_copy(edges_hbm.at[1, pl.ds(off, CHUNK)],
                              dst_idx.at[p], isem.at[1]).wait()

    idx_start(0, 0)

    zv = jnp.zeros((16,), jnp.float32)
    for r in range(ZROWS):
        for q in range(F // 16):
            zbuf[r, pl.ds(q * 16, 16)] = zv
    row0 = s * ROWS_MAIN
    for t in range(ROWS_MAIN // ZROWS):
        pltpu.async_copy(zbuf, acc.at[pl.ds(row0 + t * ZROWS, ZROWS)],
                         sem.at[0])
    for t in range(ROWS_MAIN // ZROWS):
        pltpu.make_async_copy(zbuf, acc.at[pl.ds(row0 + t * ZROWS, ZROWS)],
                              sem.at[0]).wait()

    @pl.when(s == NS - 1)
    def _():
        pltpu.sync_copy(zbuf.at[pl.ds(0, N - TAIL0)],
                        acc.at[pl.ds(TAIL0, N - TAIL0)])

    idx_wait(0, 0)
    pltpu.async_copy(feat_hbm.at[src_idx.at[0].at[0]], rows.at[0], sem.at[0])
    pltpu.async_copy(feat_hbm.at[src_idx.at[0].at[1]], rows.at[1], sem.at[1])
    plsc.subcore_barrier()

    blk0 = wid * NBLK
    rows0 = rows.at[0]
    rows1 = rows.at[1]
    sem0 = sem.at[0]
    sem1 = sem.at[1]
    NCH = NBLK // CHUNK

    for ch in range(NCH):
        p = ch % 2
        if ch + 1 < NCH:
            idx_start(ch + 1, 1 - p)
        sidx = src_idx.at[p]
        didx = dst_idx.at[p]

        def body(j, c2, sidx=sidx, didx=didx):
            i0 = 2 * j
            pltpu.make_async_copy(feat_hbm.at[sidx.at[i0]], rows0, sem0).wait()
            pltpu.sync_copy(rows0, acc.at[didx.at[i0]], add=True)
            pltpu.async_copy(feat_hbm.at[sidx.at[i0 + 2]], rows0, sem0)
            pltpu.make_async_copy(feat_hbm.at[sidx.at[i0 + 1]], rows1,
                                  sem1).wait()
            pltpu.sync_copy(rows1, acc.at[didx.at[i0 + 1]], add=True)
            pltpu.async_copy(feat_hbm.at[sidx.at[i0 + 3]], rows1, sem1)
            return c2

        lax.fori_loop(0, CHUNK // 2 - 1, body, 0)

        pltpu.make_async_copy(feat_hbm.at[sidx.at[CHUNK - 2]], rows0,
                              sem0).wait()
        pltpu.sync_copy(rows0, acc.at[didx.at[CHUNK - 2]], add=True)
        if ch + 1 < NCH:
            idx_wait(ch + 1, 1 - p)
            pltpu.async_copy(feat_hbm.at[src_idx.at[1 - p].at[0]], rows0,
                             sem0)
        pltpu.make_async_copy(feat_hbm.at[sidx.at[CHUNK - 1]], rows1,
                              sem1).wait()
        pltpu.sync_copy(rows1, acc.at[didx.at[CHUNK - 1]], add=True)
        if ch + 1 < NCH:
            pltpu.async_copy(feat_hbm.at[src_idx.at[1 - p].at[1]], rows1,
                             sem1)
    plsc.subcore_barrier()

    pltpu.sync_copy(acc.at[pl.ds(row0, ROWS_MAIN)],
                    out_hbm.at[pl.ds(c * N + row0, ROWS_MAIN)])

    @pl.when(s == NS - 1)
    def _():
        pltpu.sync_copy(acc.at[pl.ds(TAIL0, N - TAIL0)],
                        out_hbm.at[pl.ds(c * N + TAIL0, N - TAIL0)])


_sc_gather_scatter = functools.partial(
    pl.kernel,
    mesh=plsc.VectorSubcoreMesh(core_axis_name="c", subcore_axis_name="s"),
    out_type=jax.ShapeDtypeStruct((NC * N, F), jnp.float32),
    scratch_types=[
        pltpu.VMEM((2, CHUNK, B), jnp.int32),
        pltpu.VMEM((2, CHUNK, B), jnp.int32),
        pltpu.VMEM((2, B, F), jnp.float32),
        pltpu.VMEM((ZROWS, F), jnp.float32),
        pltpu.VMEM_SHARED((N, F), jnp.float32),
        pltpu.SemaphoreType.DMA((2,)),
        pltpu.SemaphoreType.DMA((2,)),
    ],
)(_sc_body)


def _feat_body(w_ref, cj_ref, o_ref):
    o_ref[...] = w_ref[...] * cj_ref[...]


def _comb_body(p_ref, ci_ref, o_ref):
    o_ref[...] = (p_ref[0] + p_ref[1]) * ci_ref[...]


def kernel(edge_index, cj, ci, u_feat, i_feat, weight):
    edges = edge_index.astype(jnp.int32).reshape(2, EBLK_TOTAL, B)

    feat = pl.pallas_call(
        _feat_body,
        out_shape=jax.ShapeDtypeStruct((N, F), jnp.float32),
    )(weight, cj)

    partials = _sc_gather_scatter(feat, edges)
    partials = partials.reshape(NC, N, F)

    rst = pl.pallas_call(
        _comb_body,
        out_shape=jax.ShapeDtypeStruct((N, F), jnp.float32),
    )(partials, ci)
    return rst
